# baseline (speedup 1.0000x reference)
import functools

import jax
import jax.numpy as jnp
from jax.experimental import pallas as pl
from jax.experimental.pallas import tpu as pltpu

T = 16384
D = 2048
TILE = 512
LANE_PAD = 128
THRESHOLD = 0.5


def _branch_route_body(x_ref, wg_ref, bg_ref, x0_ref, x1_ref, xout_ref):
    x = x_ref[...]
    z = jnp.dot(x, wg_ref[...], preferred_element_type=jnp.float32)
    z = z + bg_ref[...]
    s = 0.5 * jnp.tanh(0.5 * z) + 0.5
    a = jnp.where(s > THRESHOLD, s, 0.0)
    a0 = a[:, 0:1]
    a1 = a[:, 1:2]
    x0_ref[...] = x * a0
    x1_ref[...] = x * a1
    xout_ref[...] = x * (a0 + a1)


@jax.jit
def kernel(x, Wg, bg):
    wg_p = jnp.zeros((D, LANE_PAD), dtype=jnp.float32).at[:, :2].set(Wg)
    bg_p = jnp.zeros((1, LANE_PAD), dtype=jnp.float32).at[0, :2].set(bg)
    grid = (T // TILE,)
    out_shape = [jax.ShapeDtypeStruct((T, D), jnp.float32)] * 3
    x0, x1, xout = pl.pallas_call(
        _branch_route_body,
        grid=grid,
        in_specs=[
            pl.BlockSpec((TILE, D), lambda i: (i, 0)),
            pl.BlockSpec((D, LANE_PAD), lambda i: (0, 0)),
            pl.BlockSpec((1, LANE_PAD), lambda i: (0, 0)),
        ],
        out_specs=[pl.BlockSpec((TILE, D), lambda i: (i, 0))] * 3,
        out_shape=out_shape,
        compiler_params=pltpu.CompilerParams(
            dimension_semantics=("parallel",),
        ),
    )(x, wg_p, bg_p)
    return (x0, x1, xout)


# speedup vs baseline: 1.0022x; 1.0022x over previous
import jax
import jax.numpy as jnp
from jax.experimental import pallas as pl

T = 16384
D = 2048
TILE = 512
LANE_PAD = 128
THRESH = 0.5


def _branch_route_body(x_ref, wg_ref, bg_ref, x0_ref, x1_ref, xout_ref):
    x = x_ref[...]
    z = jnp.dot(x, wg_ref[...], preferred_element_type=jnp.float32)
    z = z + bg_ref[...]
    s = 0.5 * jnp.tanh(0.5 * z) + 0.5
    a = jnp.where(s > THRESH, s, 0.0)
    a0 = a[:, 0:1]
    a1 = a[:, 1:2]
    x0 = x * a0
    x1 = x * a1
    x0_ref[...] = x0
    x1_ref[...] = x1
    xout_ref[...] = x0 + x1


@jax.jit
def kernel(x, Wg, bg):
    wg_p = jnp.zeros((D, LANE_PAD), dtype=jnp.float32).at[:, :2].set(Wg)
    bg_p = jnp.zeros((1, LANE_PAD), dtype=jnp.float32).at[0, :2].set(bg)
    grid = (T // TILE,)
    out_shape = [jax.ShapeDtypeStruct((T, D), jnp.float32)] * 3
    x0, x1, xout = pl.pallas_call(
        _branch_route_body,
        grid=grid,
        in_specs=[
            pl.BlockSpec((TILE, D), lambda i: (i, 0)),
            pl.BlockSpec((D, LANE_PAD), lambda i: (0, 0)),
            pl.BlockSpec((1, LANE_PAD), lambda i: (0, 0)),
        ],
        out_specs=[pl.BlockSpec((TILE, D), lambda i: (i, 0))] * 3,
        out_shape=out_shape,
    )(x, wg_p, bg_p)
    return (x0, x1, xout)


# speedup vs baseline: 1.0217x; 1.0195x over previous
import jax
import jax.numpy as jnp
from jax.experimental import pallas as pl

T = 16384
D = 2048
TILE = 512
LANE_PAD = 128
THRESH = 0.5


def _branch_route_body(x_ref, wg_ref, bg_ref, x0_ref, x1_ref, xout_ref):
    x = x_ref[...]
    z = jnp.dot(x, wg_ref[...], preferred_element_type=jnp.float32)
    z = z + bg_ref[...]
    s = 0.5 * jnp.tanh(0.5 * z) + 0.5
    a = jnp.where(s > THRESH, s, 0.0)
    a0 = a[:, 0:1]
    a1 = a[:, 1:2]
    x0 = x * a0
    x1 = x * a1
    x0_ref[...] = x0
    x1_ref[...] = x1
    xout_ref[...] = x0 + x1


@jax.jit
def kernel(x, Wg, bg):
    bg_r = bg.reshape(1, 2)
    grid = (T // TILE,)
    out_shape = [jax.ShapeDtypeStruct((T, D), jnp.float32)] * 3
    x0, x1, xout = pl.pallas_call(
        _branch_route_body,
        grid=grid,
        in_specs=[
            pl.BlockSpec((TILE, D), lambda i: (i, 0)),
            pl.BlockSpec((D, 2), lambda i: (0, 0)),
            pl.BlockSpec((1, 2), lambda i: (0, 0)),
        ],
        out_specs=[pl.BlockSpec((TILE, D), lambda i: (i, 0))] * 3,
        out_shape=out_shape,
    )(x, Wg, bg_r)
    return (x0, x1, xout)
